# baseline (speedup 1.0000x reference)
import jax
import jax.numpy as jnp
import numpy as np
from jax import lax
from jax.experimental import pallas as pl
from jax.experimental.pallas import tpu as pltpu
from jax.experimental.pallas import tpu_sc as plsc

N = 50000
E = 800000
S = 5
M0 = 8
M1 = 8
NB = 8
H = 64
L = 2
RMAX = 5.0
AVG_NEI = 16.0

NP = 50176
EP = 819200
R = 128
NG = EP // R
KB = 10
NC = 2
NS = 16
ROWS_T = NP // NS
BT = 8192
PW = 32
INV_SQRT3 = 1.0 / np.sqrt(3.0)

_SC_MESH = plsc.VectorSubcoreMesh(core_axis_name="c", subcore_axis_name="s")


def _prep0_body(spec_ref, wemb_ref, wlin_ref, wrs_ref, wrg_ref, t_ref, r_ref):
    spec = spec_ref[...]
    attr = (spec == lax.broadcasted_iota(jnp.int32, (spec.shape[0], S), 1))
    attr = attr.astype(jnp.float32)
    h_s = attr @ wemb_ref[...]
    t_ref[...] = jnp.concatenate(
        [h_s @ wlin_ref[...] / np.sqrt(M0),
         jnp.zeros((h_s.shape[0], 3 * M1), jnp.float32)], axis=1)
    res_s = jnp.zeros((h_s.shape[0], M0), jnp.float32)
    res_g = jnp.zeros((h_s.shape[0], M1), jnp.float32)
    for k in range(S):
        mk = attr[:, k:k + 1]
        res_s = res_s + (h_s @ wrs_ref[k]) * mk
        res_g = res_g + (h_s @ wrg_ref[k]) * mk
    r_ref[...] = jnp.concatenate(
        [res_s / np.sqrt(M0 * S), res_g / np.sqrt(M0 * S)], axis=1)


def _node_prep0(spec_p, W_embed, W_lin_s0, wrs0, wrg0):
    BN = NP // 64
    return pl.pallas_call(
        _prep0_body,
        grid=(64,),
        in_specs=[
            pl.BlockSpec((BN, 1), lambda i: (i, 0)),
            pl.BlockSpec((S, M0), lambda i: (0, 0)),
            pl.BlockSpec((M0, M0), lambda i: (0, 0)),
            pl.BlockSpec((S, M0, M0), lambda i: (0, 0, 0)),
            pl.BlockSpec((S, M0, M1), lambda i: (0, 0, 0)),
        ],
        out_specs=[
            pl.BlockSpec((BN, 4 * M0), lambda i: (i, 0)),
            pl.BlockSpec((BN, 2 * M0), lambda i: (i, 0)),
        ],
        out_shape=[
            jax.ShapeDtypeStruct((NP, 4 * M0), jnp.float32),
            jax.ShapeDtypeStruct((NP, 2 * M0), jnp.float32),
        ],
    )(spec_p, W_embed, W_lin_s0, wrs0, wrg0)


def _gather_body(t_ref, idx_ref, out_ref, idxb, rows, isem0, isem1,
                 gsem, ssem0, ssem1):
    c = lax.axis_index("c")
    s = lax.axis_index("s")
    wid = s * NC + c
    gpw = NG // (NC * NS)
    nit = gpw // KB
    isem = (isem0, isem1)
    ssem = (ssem0, ssem1)

    pltpu.async_copy(idx_ref.at[pl.ds(wid * gpw, KB)], idxb.at[0], isem0)

    def body(jj, _):
        for par in range(2):
            j = jj * 2 + par
            gb = wid * gpw + j * KB
            pltpu.make_async_copy(
                idx_ref.at[pl.ds(0, KB)], idxb.at[par], isem[par]).wait()

            @pl.when(j >= 2)
            def _():
                pltpu.make_async_copy(
                    rows.at[par], out_ref.at[pl.ds(0, KB)], ssem[par]).wait()

            @pl.when(j + 1 < nit)
            def _():
                pltpu.async_copy(idx_ref.at[pl.ds(gb + KB, KB)],
                                 idxb.at[1 - par], isem[1 - par])

            descs = [pltpu.async_copy(t_ref.at[idxb.at[par, b]],
                                      rows.at[par, b], gsem)
                     for b in range(KB)]
            for d in descs:
                d.wait()
            pltpu.async_copy(rows.at[par], out_ref.at[pl.ds(gb, KB)],
                             ssem[par])
        return 0

    lax.fori_loop(0, nit // 2, body, 0)
    pltpu.make_async_copy(rows.at[0], out_ref.at[pl.ds(0, KB)], ssem0).wait()
    pltpu.make_async_copy(rows.at[1], out_ref.at[pl.ds(0, KB)], ssem1).wait()


_gather = pl.kernel(
    _gather_body,
    out_type=jax.ShapeDtypeStruct((NG, R, 4 * M0), jnp.float32),
    mesh=_SC_MESH,
    scratch_types=[
        pltpu.VMEM((2, KB, R), jnp.int32),
        pltpu.VMEM((2, KB, R, 4 * M0), jnp.float32),
        pltpu.SemaphoreType.DMA,
        pltpu.SemaphoreType.DMA,
        pltpu.SemaphoreType.DMA,
        pltpu.SemaphoreType.DMA,
        pltpu.SemaphoreType.DMA,
    ],
    compiler_params=pltpu.CompilerParams(use_tc_tiling_on_sc=False),
)


def _w_body(evt_ref, wm1t_ref, wm2t_ref, wm3t_ref, w_ref):
    vt = evt_ref[...]
    vx = vt[0:1]
    vy = vt[1:2]
    vz = vt[2:3]
    r = jnp.sqrt(vx * vx + vy * vy + vz * vz + 1e-12)
    x = r / RMAX
    x2 = x * x
    x4 = x2 * x2
    x5 = x4 * x
    env = 1.0 - 21.0 * x5 + 35.0 * x5 * x - 15.0 * x5 * x2
    env = jnp.where(x < 1.0, env, 0.0)
    nvec = lax.broadcasted_iota(jnp.int32, (NB, 1), 0).astype(jnp.float32) + 1.0
    bess = np.sqrt(2.0 / RMAX) * jnp.sin(nvec * np.pi * r / RMAX) / r
    emb = bess * env

    h1 = jax.nn.silu(wm1t_ref[...] @ emb)
    h2 = jax.nn.silu(wm2t_ref[...] @ h1)
    w_ref[...] = wm3t_ref[...] @ h2


def _w_stage(evt_p, wm1t, wm2t, wm3t):
    nblk = EP // BT
    return pl.pallas_call(
        _w_body,
        grid=(nblk,),
        in_specs=[
            pl.BlockSpec((3, BT), lambda i: (0, i)),
            pl.BlockSpec((H, NB), lambda i: (0, 0)),
            pl.BlockSpec((H, H), lambda i: (0, 0)),
            pl.BlockSpec((4 * M0, H), lambda i: (0, 0)),
        ],
        out_specs=pl.BlockSpec((4 * M0, BT), lambda i: (0, i)),
        out_shape=jax.ShapeDtypeStruct((4 * M0, EP), jnp.float32),
    )(evt_p, wm1t, wm2t, wm3t)


def _edge_body(evt_ref, g_ref, w_ref, q_ref):
    vt = evt_ref[...]
    vx = vt[0:1]
    vy = vt[1:2]
    vz = vt[2:3]
    r = jnp.sqrt(vx * vx + vy * vy + vz * vz + 1e-12)
    yx = vx / r
    yy = vy / r
    yz = vz / r
    w = w_ref[...]

    gp = g_ref[...]
    gt = jnp.concatenate(
        [jnp.transpose(gp[:, 32 * qq:32 * qq + 32]) for qq in range(4)],
        axis=1)
    es = gt[0:8]
    evx = gt[8:16]
    evy = gt[16:24]
    evz = gt[24:32]

    dot = evx * yx + evy * yy + evz * yz
    m_s = jnp.concatenate([w[0:8] * es, w[24:32] * dot * INV_SQRT3],
                          axis=0)
    a = w[8:16] * es
    w3 = w[16:24]
    mvx = jnp.concatenate([a * yx, w3 * evx], axis=0)
    mvy = jnp.concatenate([a * yy, w3 * evy], axis=0)
    mvz = jnp.concatenate([a * yz, w3 * evz], axis=0)

    qt0 = jnp.concatenate([m_s, mvx], axis=0)
    qt1 = jnp.concatenate([mvy, mvz], axis=0)
    B4 = BT // 4
    q_ref[0] = jnp.concatenate(
        [jnp.transpose(qt0[:, qq * B4:(qq + 1) * B4]) for qq in range(4)],
        axis=1)
    q_ref[1] = jnp.concatenate(
        [jnp.transpose(qt1[:, qq * B4:(qq + 1) * B4]) for qq in range(4)],
        axis=1)


def _edge_stage(evt_p, g_flat, w_all):
    nblk = EP // BT
    return pl.pallas_call(
        _edge_body,
        grid=(nblk,),
        in_specs=[
            pl.BlockSpec((3, BT), lambda i: (0, i)),
            pl.BlockSpec((BT // 4, 128), lambda i: (i, 0)),
            pl.BlockSpec((4 * M0, BT), lambda i: (0, i)),
        ],
        out_specs=pl.BlockSpec((NC, BT // 4, 128), lambda i: (0, i, 0)),
        out_shape=jax.ShapeDtypeStruct((NC, EP // 4, 128), jnp.float32),
    )(evt_p, g_flat, w_all)


KB2 = 5


def _scatter_body(q_ref, ridx_ref, z_ref, out_ref, idxb, buf, table, sem):
    c = lax.axis_index("c")
    s = lax.axis_index("s")
    r0 = s * ROWS_T

    pltpu.sync_copy(z_ref.at[pl.ds(r0, ROWS_T)], table.at[pl.ds(r0, ROWS_T)])
    plsc.subcore_barrier()

    gpw = NG // NS

    def body(j, _):
        gb = s * gpw + j * KB2
        pltpu.sync_copy(ridx_ref.at[pl.ds(gb, KB2)], idxb)
        pltpu.sync_copy(q_ref.at[c, pl.ds(gb, KB2)], buf)
        for b in range(KB2):
            pltpu.sync_copy(buf.at[b], table.at[idxb.at[b]], add=True)
        return 0

    lax.fori_loop(0, gpw // KB2, body, 0)
    plsc.subcore_barrier()

    pltpu.sync_copy(table.at[pl.ds(r0, ROWS_T)],
                    out_ref.at[c, pl.ds(r0, ROWS_T)])


_scatter = pl.kernel(
    _scatter_body,
    out_type=jax.ShapeDtypeStruct((NC, NP, PW), jnp.float32),
    mesh=_SC_MESH,
    scratch_types=[
        pltpu.VMEM((KB2, R), jnp.int32),
        pltpu.VMEM((KB2, R, PW), jnp.float32),
        pltpu.VMEM_SHARED((NP, PW), jnp.float32),
        pltpu.SemaphoreType.DMA,
    ],
    compiler_params=pltpu.CompilerParams(use_tc_tiling_on_sc=False),
)


def _update_core(agg_ref, r_ref, rv_ref, wos_ref, wov_ref):
    agg0 = agg_ref[0]
    agg1 = agg_ref[1]
    inv4 = 1.0 / np.sqrt(M0 + M1)
    agg_s = agg0[:, 0:16] / AVG_NEI
    out_s = (agg_s @ wos_ref[...]) * inv4 + r_ref[...]
    wov = wov_ref[...]
    rv = rv_ref[...]
    out_vx = ((agg0[:, 16:32] / AVG_NEI) @ wov) * inv4 + rv[:, 0:8]
    out_vy = ((agg1[:, 0:16] / AVG_NEI) @ wov) * inv4 + rv[:, 8:16]
    out_vz = ((agg1[:, 16:32] / AVG_NEI) @ wov) * inv4 + rv[:, 16:24]
    hs = jax.nn.silu(out_s[:, 0:M0])
    gate = jax.nn.silu(out_s[:, M0:2 * M0])
    return hs, out_vx * gate, out_vy * gate, out_vz * gate


def _update_mid_body(agg_ref, r_ref, rv_ref, spec_ref, wos_ref, wov_ref,
                     wls_ref, wlv_ref, wrs_ref, wrg_ref, wrv_ref,
                     t_ref, rn_ref, rvn_ref):
    hs, vx, vy, vz = _update_core(agg_ref, r_ref, rv_ref, wos_ref, wov_ref)
    spec = spec_ref[...]
    attr = (spec == lax.broadcasted_iota(jnp.int32, (spec.shape[0], S), 1))
    attr = attr.astype(jnp.float32)
    res_s = jnp.zeros_like(hs)
    res_g = jnp.zeros_like(hs)
    res_vx = jnp.zeros_like(hs)
    res_vy = jnp.zeros_like(hs)
    res_vz = jnp.zeros_like(hs)
    for k in range(S):
        mk = attr[:, k:k + 1]
        res_s = res_s + (hs @ wrs_ref[k]) * mk
        res_g = res_g + (hs @ wrg_ref[k]) * mk
        res_vx = res_vx + (vx @ wrv_ref[k]) * mk
        res_vy = res_vy + (vy @ wrv_ref[k]) * mk
        res_vz = res_vz + (vz @ wrv_ref[k]) * mk
    inv_s = 1.0 / np.sqrt(M0 * S)
    inv_v = 1.0 / np.sqrt(M1 * S)
    rn_ref[...] = jnp.concatenate([res_s * inv_s, res_g * inv_s], axis=1)
    rvn_ref[...] = jnp.concatenate(
        [res_vx * inv_v, res_vy * inv_v, res_vz * inv_v], axis=1)
    wl_s = wls_ref[...]
    wl_v = wlv_ref[...]
    t_ref[...] = jnp.concatenate(
        [hs @ wl_s / np.sqrt(M0), vx @ wl_v / np.sqrt(M1),
         vy @ wl_v / np.sqrt(M1), vz @ wl_v / np.sqrt(M1)], axis=1)


def _update_last_body(agg_ref, r_ref, rv_ref, wos_ref, wov_ref,
                      wf1_ref, wf2_ref, e_ref):
    hs, _, _, _ = _update_core(agg_ref, r_ref, rv_ref, wos_ref, wov_ref)
    e_ref[...] = (hs @ wf1_ref[...]) @ wf2_ref[...]


def _node_update_mid(agg, rcur, rvcur, spec_p, wos, wov, wls, wlv,
                     wrs, wrg, wrv):
    BN = NP // 64
    return pl.pallas_call(
        _update_mid_body,
        grid=(64,),
        in_specs=[
            pl.BlockSpec((NC, BN, PW), lambda i: (0, i, 0)),
            pl.BlockSpec((BN, 16), lambda i: (i, 0)),
            pl.BlockSpec((BN, 24), lambda i: (i, 0)),
            pl.BlockSpec((BN, 1), lambda i: (i, 0)),
            pl.BlockSpec((2 * M0, 2 * M0), lambda i: (0, 0)),
            pl.BlockSpec((2 * M0, M1), lambda i: (0, 0)),
            pl.BlockSpec((M0, M0), lambda i: (0, 0)),
            pl.BlockSpec((M1, M1), lambda i: (0, 0)),
            pl.BlockSpec((S, M0, M0), lambda i: (0, 0, 0)),
            pl.BlockSpec((S, M0, M1), lambda i: (0, 0, 0)),
            pl.BlockSpec((S, M1, M1), lambda i: (0, 0, 0)),
        ],
        out_specs=[
            pl.BlockSpec((BN, 4 * M0), lambda i: (i, 0)),
            pl.BlockSpec((BN, 2 * M0), lambda i: (i, 0)),
            pl.BlockSpec((BN, 3 * M1), lambda i: (i, 0)),
        ],
        out_shape=[
            jax.ShapeDtypeStruct((NP, 4 * M0), jnp.float32),
            jax.ShapeDtypeStruct((NP, 2 * M0), jnp.float32),
            jax.ShapeDtypeStruct((NP, 3 * M1), jnp.float32),
        ],
    )(agg, rcur, rvcur, spec_p, wos, wov, wls, wlv, wrs, wrg, wrv)


def _node_update_last(agg, rcur, rvcur, wos, wov, wf1, wf2):
    BN = NP // 64
    return pl.pallas_call(
        _update_last_body,
        grid=(64,),
        in_specs=[
            pl.BlockSpec((NC, BN, PW), lambda i: (0, i, 0)),
            pl.BlockSpec((BN, 16), lambda i: (i, 0)),
            pl.BlockSpec((BN, 24), lambda i: (i, 0)),
            pl.BlockSpec((2 * M0, 2 * M0), lambda i: (0, 0)),
            pl.BlockSpec((2 * M0, M1), lambda i: (0, 0)),
            pl.BlockSpec((M0, M0 // 2), lambda i: (0, 0)),
            pl.BlockSpec((M0 // 2, 1), lambda i: (0, 0)),
        ],
        out_specs=pl.BlockSpec((BN, 1), lambda i: (i, 0)),
        out_shape=jax.ShapeDtypeStruct((NP, 1), jnp.float32),
    )(agg, rcur, rvcur, wos, wov, wf1, wf2)


def kernel(edge_vectors, node_species, senders, receivers, W_embed, W_res_s,
           W_res_g, W_res_v, W_lin_s, W_lin_v, Wm1, Wm2, Wm3, W_out_s,
           W_out_v, W_f1, W_f2):
    f32 = jnp.float32
    spec_p = jnp.concatenate(
        [node_species.astype(jnp.int32), jnp.zeros((NP - N,), jnp.int32)]
    ).reshape(NP, 1)
    pad_vec = jnp.broadcast_to(jnp.array([2.0 * RMAX, 0.0, 0.0], f32),
                               (EP - E, 3))
    evt_p = jnp.concatenate([edge_vectors, pad_vec], axis=0).T
    def _interleave(a):
        return (a.reshape(EP // BT, 4, BT // 4).transpose(0, 2, 1)
                .reshape(NG, R))

    send_p = _interleave(jnp.concatenate(
        [senders.astype(jnp.int32), jnp.zeros((EP - E,), jnp.int32)]))
    recv_p = _interleave(jnp.concatenate(
        [receivers.astype(jnp.int32), jnp.zeros((EP - E,), jnp.int32)]))
    zeros_tab = jnp.zeros((NP, PW), f32)

    wrs = [W_res_s[l].transpose(1, 0, 2) for l in range(L)]
    wrg = [W_res_g[l].transpose(1, 0, 2) for l in range(L)]
    wrv = [W_res_v[l].transpose(1, 0, 2) for l in range(L)]

    t, rcur = _node_prep0(spec_p, W_embed, W_lin_s[0], wrs[0], wrg[0])
    rvcur = jnp.zeros((NP, 3 * M1), f32)

    w0 = _w_stage(evt_p, Wm1[0].T, Wm2[0].T, Wm3[0].T)
    wlist = [w0, None]

    for l in range(L):
        g = _gather(t, send_p).reshape(EP // 4, 128)
        q = _edge_stage(evt_p, g, wlist[l])
        if l == 0:
            wlist[1] = _w_stage(evt_p, Wm1[1].T, Wm2[1].T, Wm3[1].T)
        agg = _scatter(q.reshape(NC, NG, R, PW), recv_p, zeros_tab)
        if l < L - 1:
            t, rcur, rvcur = _node_update_mid(
                agg, rcur, rvcur, spec_p, W_out_s[l], W_out_v[l],
                W_lin_s[l + 1], W_lin_v[l + 1],
                wrs[l + 1], wrg[l + 1], wrv[l + 1])
        else:
            e = _node_update_last(agg, rcur, rvcur, W_out_s[l], W_out_v[l],
                                  W_f1, W_f2)
    return e[:N, 0]


# speedup vs baseline: 1.0318x; 1.0318x over previous
import jax
import jax.numpy as jnp
import numpy as np
from jax import lax
from jax.experimental import pallas as pl
from jax.experimental.pallas import tpu as pltpu
from jax.experimental.pallas import tpu_sc as plsc

N = 50000
E = 800000
S = 5
M0 = 8
M1 = 8
NB = 8
H = 64
L = 2
RMAX = 5.0
AVG_NEI = 16.0

NP = 50176
EP = 819200
R = 128
NG = EP // R
KB = 10
NC = 2
NS = 16
ROWS_T = NP // NS
BT = 8192
PW = 32
INV_SQRT3 = 1.0 / np.sqrt(3.0)

_SC_MESH = plsc.VectorSubcoreMesh(core_axis_name="c", subcore_axis_name="s")


def _prep0_body(spec_ref, wemb_ref, wlin_ref, wrs_ref, wrg_ref, t_ref, r_ref):
    spec = spec_ref[...]
    attr = (spec == lax.broadcasted_iota(jnp.int32, (spec.shape[0], S), 1))
    attr = attr.astype(jnp.float32)
    h_s = attr @ wemb_ref[...]
    t_ref[...] = jnp.concatenate(
        [h_s @ wlin_ref[...] / np.sqrt(M0),
         jnp.zeros((h_s.shape[0], 3 * M1), jnp.float32)], axis=1)
    res_s = jnp.zeros((h_s.shape[0], M0), jnp.float32)
    res_g = jnp.zeros((h_s.shape[0], M1), jnp.float32)
    for k in range(S):
        mk = attr[:, k:k + 1]
        res_s = res_s + (h_s @ wrs_ref[k]) * mk
        res_g = res_g + (h_s @ wrg_ref[k]) * mk
    r_ref[...] = jnp.concatenate(
        [res_s / np.sqrt(M0 * S), res_g / np.sqrt(M0 * S)], axis=1)


def _node_prep0(spec_p, W_embed, W_lin_s0, wrs0, wrg0):
    BN = NP // 64
    return pl.pallas_call(
        _prep0_body,
        grid=(64,),
        in_specs=[
            pl.BlockSpec((BN, 1), lambda i: (i, 0)),
            pl.BlockSpec((S, M0), lambda i: (0, 0)),
            pl.BlockSpec((M0, M0), lambda i: (0, 0)),
            pl.BlockSpec((S, M0, M0), lambda i: (0, 0, 0)),
            pl.BlockSpec((S, M0, M1), lambda i: (0, 0, 0)),
        ],
        out_specs=[
            pl.BlockSpec((BN, 4 * M0), lambda i: (i, 0)),
            pl.BlockSpec((BN, 2 * M0), lambda i: (i, 0)),
        ],
        out_shape=[
            jax.ShapeDtypeStruct((NP, 4 * M0), jnp.float32),
            jax.ShapeDtypeStruct((NP, 2 * M0), jnp.float32),
        ],
    )(spec_p, W_embed, W_lin_s0, wrs0, wrg0)


def _gather_body(t_ref, idx_ref, out_ref, idxb, rows, isem0, isem1,
                 gsem, ssem0, ssem1):
    c = lax.axis_index("c")
    s = lax.axis_index("s")
    wid = s * NC + c
    gpw = NG // (NC * NS)
    nit = gpw // KB
    isem = (isem0, isem1)
    ssem = (ssem0, ssem1)

    pltpu.async_copy(idx_ref.at[pl.ds(wid * gpw, KB)], idxb.at[0], isem0)

    def body(jj, _):
        for par in range(2):
            j = jj * 2 + par
            gb = wid * gpw + j * KB
            pltpu.make_async_copy(
                idx_ref.at[pl.ds(0, KB)], idxb.at[par], isem[par]).wait()

            @pl.when(j >= 2)
            def _():
                pltpu.make_async_copy(
                    rows.at[par], out_ref.at[pl.ds(0, KB)], ssem[par]).wait()

            @pl.when(j + 1 < nit)
            def _():
                pltpu.async_copy(idx_ref.at[pl.ds(gb + KB, KB)],
                                 idxb.at[1 - par], isem[1 - par])

            descs = [pltpu.async_copy(t_ref.at[idxb.at[par, b]],
                                      rows.at[par, b], gsem)
                     for b in range(KB)]
            for d in descs:
                d.wait()
            pltpu.async_copy(rows.at[par], out_ref.at[pl.ds(gb, KB)],
                             ssem[par])
        return 0

    lax.fori_loop(0, nit // 2, body, 0)
    pltpu.make_async_copy(rows.at[0], out_ref.at[pl.ds(0, KB)], ssem0).wait()
    pltpu.make_async_copy(rows.at[1], out_ref.at[pl.ds(0, KB)], ssem1).wait()


_gather = pl.kernel(
    _gather_body,
    out_type=jax.ShapeDtypeStruct((NG, R, 4 * M0), jnp.float32),
    mesh=_SC_MESH,
    scratch_types=[
        pltpu.VMEM((2, KB, R), jnp.int32),
        pltpu.VMEM((2, KB, R, 4 * M0), jnp.float32),
        pltpu.SemaphoreType.DMA,
        pltpu.SemaphoreType.DMA,
        pltpu.SemaphoreType.DMA,
        pltpu.SemaphoreType.DMA,
        pltpu.SemaphoreType.DMA,
    ],
    compiler_params=pltpu.CompilerParams(use_tc_tiling_on_sc=False),
)


def _w_body(evt_ref, wm1t_ref, wm2t_ref, wm3t_ref, w_ref):
    vt = evt_ref[...]
    vx = vt[0:1]
    vy = vt[1:2]
    vz = vt[2:3]
    r = jnp.sqrt(vx * vx + vy * vy + vz * vz + 1e-12)
    x = r / RMAX
    x2 = x * x
    x4 = x2 * x2
    x5 = x4 * x
    env = 1.0 - 21.0 * x5 + 35.0 * x5 * x - 15.0 * x5 * x2
    env = jnp.where(x < 1.0, env, 0.0)
    nvec = lax.broadcasted_iota(jnp.int32, (NB, 1), 0).astype(jnp.float32) + 1.0
    bess = np.sqrt(2.0 / RMAX) * jnp.sin(nvec * np.pi * r / RMAX) / r
    emb = bess * env

    h1 = jax.nn.silu(wm1t_ref[...] @ emb)
    h2 = jax.nn.silu(wm2t_ref[...] @ h1)
    w_ref[...] = wm3t_ref[...] @ h2


def _w_stage(evt_p, wm1t, wm2t, wm3t):
    nblk = EP // BT
    return pl.pallas_call(
        _w_body,
        grid=(nblk,),
        in_specs=[
            pl.BlockSpec((3, BT), lambda i: (0, i)),
            pl.BlockSpec((H, NB), lambda i: (0, 0)),
            pl.BlockSpec((H, H), lambda i: (0, 0)),
            pl.BlockSpec((4 * M0, H), lambda i: (0, 0)),
        ],
        out_specs=pl.BlockSpec((4 * M0, BT), lambda i: (0, i)),
        out_shape=jax.ShapeDtypeStruct((4 * M0, EP), jnp.float32),
    )(evt_p, wm1t, wm2t, wm3t)


def _edge_body(evt_ref, g_ref, w_ref, q_ref):
    vt = evt_ref[...]
    vx = vt[0:1]
    vy = vt[1:2]
    vz = vt[2:3]
    r = jnp.sqrt(vx * vx + vy * vy + vz * vz + 1e-12)
    yx = vx / r
    yy = vy / r
    yz = vz / r
    w = w_ref[...]

    gp = g_ref[...]
    gt = jnp.concatenate(
        [jnp.transpose(gp[:, 32 * qq:32 * qq + 32]) for qq in range(4)],
        axis=1)
    es = gt[0:8]
    evx = gt[8:16]
    evy = gt[16:24]
    evz = gt[24:32]

    dot = evx * yx + evy * yy + evz * yz
    m_s = jnp.concatenate([w[0:8] * es, w[24:32] * dot * INV_SQRT3],
                          axis=0)
    a = w[8:16] * es
    w3 = w[16:24]
    mvx = jnp.concatenate([a * yx, w3 * evx], axis=0)
    mvy = jnp.concatenate([a * yy, w3 * evy], axis=0)
    mvz = jnp.concatenate([a * yz, w3 * evz], axis=0)

    qt0 = jnp.concatenate([m_s, mvx], axis=0)
    qt1 = jnp.concatenate([mvy, mvz], axis=0)
    B4 = BT // 4
    q_ref[0] = jnp.concatenate(
        [jnp.transpose(qt0[:, qq * B4:(qq + 1) * B4]) for qq in range(4)],
        axis=1)
    q_ref[1] = jnp.concatenate(
        [jnp.transpose(qt1[:, qq * B4:(qq + 1) * B4]) for qq in range(4)],
        axis=1)


def _edge_stage(evt_p, g_flat, w_all):
    nblk = EP // BT
    return pl.pallas_call(
        _edge_body,
        grid=(nblk,),
        in_specs=[
            pl.BlockSpec((3, BT), lambda i: (0, i)),
            pl.BlockSpec((BT // 4, 128), lambda i: (i, 0)),
            pl.BlockSpec((4 * M0, BT), lambda i: (0, i)),
        ],
        out_specs=pl.BlockSpec((NC, BT // 4, 128), lambda i: (0, i, 0)),
        out_shape=jax.ShapeDtypeStruct((NC, EP // 4, 128), jnp.float32),
    )(evt_p, g_flat, w_all)


KB2 = 2


def _scatter_body(q_ref, ridx_ref, z_ref, out_ref, idxb, buf, table,
                  li0, li1, lq0, lq1):
    c = lax.axis_index("c")
    s = lax.axis_index("s")
    r0 = s * ROWS_T
    li = (li0, li1)
    lq = (lq0, lq1)

    pltpu.sync_copy(z_ref.at[pl.ds(r0, ROWS_T)], table.at[pl.ds(r0, ROWS_T)])
    plsc.subcore_barrier()

    gpw = NG // NS
    nit = gpw // KB2
    gb0 = s * gpw

    pltpu.async_copy(ridx_ref.at[pl.ds(gb0, KB2)], idxb.at[0], li0)
    pltpu.async_copy(q_ref.at[c, pl.ds(gb0, KB2)], buf.at[0], lq0)

    def body(jj, _):
        for par in range(2):
            j = jj * 2 + par
            gb = gb0 + j * KB2
            pltpu.make_async_copy(
                ridx_ref.at[pl.ds(0, KB2)], idxb.at[par], li[par]).wait()
            pltpu.make_async_copy(
                q_ref.at[c, pl.ds(0, KB2)], buf.at[par], lq[par]).wait()

            @pl.when(j + 1 < nit)
            def _():
                pltpu.async_copy(ridx_ref.at[pl.ds(gb + KB2, KB2)],
                                 idxb.at[1 - par], li[1 - par])
                pltpu.async_copy(q_ref.at[c, pl.ds(gb + KB2, KB2)],
                                 buf.at[1 - par], lq[1 - par])

            for b in range(KB2):
                pltpu.sync_copy(buf.at[par, b], table.at[idxb.at[par, b]],
                                add=True)
        return 0

    lax.fori_loop(0, nit // 2, body, 0)
    plsc.subcore_barrier()

    pltpu.sync_copy(table.at[pl.ds(r0, ROWS_T)],
                    out_ref.at[c, pl.ds(r0, ROWS_T)])


_scatter = pl.kernel(
    _scatter_body,
    out_type=jax.ShapeDtypeStruct((NC, NP, PW), jnp.float32),
    mesh=_SC_MESH,
    scratch_types=[
        pltpu.VMEM((2, KB2, R), jnp.int32),
        pltpu.VMEM((2, KB2, R, PW), jnp.float32),
        pltpu.VMEM_SHARED((NP, PW), jnp.float32),
        pltpu.SemaphoreType.DMA,
        pltpu.SemaphoreType.DMA,
        pltpu.SemaphoreType.DMA,
        pltpu.SemaphoreType.DMA,
    ],
    compiler_params=pltpu.CompilerParams(use_tc_tiling_on_sc=False),
)


def _update_core(agg_ref, r_ref, rv_ref, wos_ref, wov_ref):
    agg0 = agg_ref[0]
    agg1 = agg_ref[1]
    inv4 = 1.0 / np.sqrt(M0 + M1)
    agg_s = agg0[:, 0:16] / AVG_NEI
    out_s = (agg_s @ wos_ref[...]) * inv4 + r_ref[...]
    wov = wov_ref[...]
    rv = rv_ref[...]
    out_vx = ((agg0[:, 16:32] / AVG_NEI) @ wov) * inv4 + rv[:, 0:8]
    out_vy = ((agg1[:, 0:16] / AVG_NEI) @ wov) * inv4 + rv[:, 8:16]
    out_vz = ((agg1[:, 16:32] / AVG_NEI) @ wov) * inv4 + rv[:, 16:24]
    hs = jax.nn.silu(out_s[:, 0:M0])
    gate = jax.nn.silu(out_s[:, M0:2 * M0])
    return hs, out_vx * gate, out_vy * gate, out_vz * gate


def _update_mid_body(agg_ref, r_ref, rv_ref, spec_ref, wos_ref, wov_ref,
                     wls_ref, wlv_ref, wrs_ref, wrg_ref, wrv_ref,
                     t_ref, rn_ref, rvn_ref):
    hs, vx, vy, vz = _update_core(agg_ref, r_ref, rv_ref, wos_ref, wov_ref)
    spec = spec_ref[...]
    attr = (spec == lax.broadcasted_iota(jnp.int32, (spec.shape[0], S), 1))
    attr = attr.astype(jnp.float32)
    res_s = jnp.zeros_like(hs)
    res_g = jnp.zeros_like(hs)
    res_vx = jnp.zeros_like(hs)
    res_vy = jnp.zeros_like(hs)
    res_vz = jnp.zeros_like(hs)
    for k in range(S):
        mk = attr[:, k:k + 1]
        res_s = res_s + (hs @ wrs_ref[k]) * mk
        res_g = res_g + (hs @ wrg_ref[k]) * mk
        res_vx = res_vx + (vx @ wrv_ref[k]) * mk
        res_vy = res_vy + (vy @ wrv_ref[k]) * mk
        res_vz = res_vz + (vz @ wrv_ref[k]) * mk
    inv_s = 1.0 / np.sqrt(M0 * S)
    inv_v = 1.0 / np.sqrt(M1 * S)
    rn_ref[...] = jnp.concatenate([res_s * inv_s, res_g * inv_s], axis=1)
    rvn_ref[...] = jnp.concatenate(
        [res_vx * inv_v, res_vy * inv_v, res_vz * inv_v], axis=1)
    wl_s = wls_ref[...]
    wl_v = wlv_ref[...]
    t_ref[...] = jnp.concatenate(
        [hs @ wl_s / np.sqrt(M0), vx @ wl_v / np.sqrt(M1),
         vy @ wl_v / np.sqrt(M1), vz @ wl_v / np.sqrt(M1)], axis=1)


def _update_last_body(agg_ref, r_ref, rv_ref, wos_ref, wov_ref,
                      wf1_ref, wf2_ref, e_ref):
    hs, _, _, _ = _update_core(agg_ref, r_ref, rv_ref, wos_ref, wov_ref)
    e_ref[...] = (hs @ wf1_ref[...]) @ wf2_ref[...]


def _node_update_mid(agg, rcur, rvcur, spec_p, wos, wov, wls, wlv,
                     wrs, wrg, wrv):
    BN = NP // 64
    return pl.pallas_call(
        _update_mid_body,
        grid=(64,),
        in_specs=[
            pl.BlockSpec((NC, BN, PW), lambda i: (0, i, 0)),
            pl.BlockSpec((BN, 16), lambda i: (i, 0)),
            pl.BlockSpec((BN, 24), lambda i: (i, 0)),
            pl.BlockSpec((BN, 1), lambda i: (i, 0)),
            pl.BlockSpec((2 * M0, 2 * M0), lambda i: (0, 0)),
            pl.BlockSpec((2 * M0, M1), lambda i: (0, 0)),
            pl.BlockSpec((M0, M0), lambda i: (0, 0)),
            pl.BlockSpec((M1, M1), lambda i: (0, 0)),
            pl.BlockSpec((S, M0, M0), lambda i: (0, 0, 0)),
            pl.BlockSpec((S, M0, M1), lambda i: (0, 0, 0)),
            pl.BlockSpec((S, M1, M1), lambda i: (0, 0, 0)),
        ],
        out_specs=[
            pl.BlockSpec((BN, 4 * M0), lambda i: (i, 0)),
            pl.BlockSpec((BN, 2 * M0), lambda i: (i, 0)),
            pl.BlockSpec((BN, 3 * M1), lambda i: (i, 0)),
        ],
        out_shape=[
            jax.ShapeDtypeStruct((NP, 4 * M0), jnp.float32),
            jax.ShapeDtypeStruct((NP, 2 * M0), jnp.float32),
            jax.ShapeDtypeStruct((NP, 3 * M1), jnp.float32),
        ],
    )(agg, rcur, rvcur, spec_p, wos, wov, wls, wlv, wrs, wrg, wrv)


def _node_update_last(agg, rcur, rvcur, wos, wov, wf1, wf2):
    BN = NP // 64
    return pl.pallas_call(
        _update_last_body,
        grid=(64,),
        in_specs=[
            pl.BlockSpec((NC, BN, PW), lambda i: (0, i, 0)),
            pl.BlockSpec((BN, 16), lambda i: (i, 0)),
            pl.BlockSpec((BN, 24), lambda i: (i, 0)),
            pl.BlockSpec((2 * M0, 2 * M0), lambda i: (0, 0)),
            pl.BlockSpec((2 * M0, M1), lambda i: (0, 0)),
            pl.BlockSpec((M0, M0 // 2), lambda i: (0, 0)),
            pl.BlockSpec((M0 // 2, 1), lambda i: (0, 0)),
        ],
        out_specs=pl.BlockSpec((BN, 1), lambda i: (i, 0)),
        out_shape=jax.ShapeDtypeStruct((NP, 1), jnp.float32),
    )(agg, rcur, rvcur, wos, wov, wf1, wf2)


def kernel(edge_vectors, node_species, senders, receivers, W_embed, W_res_s,
           W_res_g, W_res_v, W_lin_s, W_lin_v, Wm1, Wm2, Wm3, W_out_s,
           W_out_v, W_f1, W_f2):
    f32 = jnp.float32
    spec_p = jnp.concatenate(
        [node_species.astype(jnp.int32), jnp.zeros((NP - N,), jnp.int32)]
    ).reshape(NP, 1)
    pad_vec = jnp.broadcast_to(jnp.array([2.0 * RMAX, 0.0, 0.0], f32),
                               (EP - E, 3))
    evt_p = jnp.concatenate([edge_vectors, pad_vec], axis=0).T
    def _interleave(a):
        return (a.reshape(EP // BT, 4, BT // 4).transpose(0, 2, 1)
                .reshape(NG, R))

    send_p = _interleave(jnp.concatenate(
        [senders.astype(jnp.int32), jnp.zeros((EP - E,), jnp.int32)]))
    recv_p = _interleave(jnp.concatenate(
        [receivers.astype(jnp.int32), jnp.zeros((EP - E,), jnp.int32)]))
    zeros_tab = jnp.zeros((NP, PW), f32)

    wrs = [W_res_s[l].transpose(1, 0, 2) for l in range(L)]
    wrg = [W_res_g[l].transpose(1, 0, 2) for l in range(L)]
    wrv = [W_res_v[l].transpose(1, 0, 2) for l in range(L)]

    t, rcur = _node_prep0(spec_p, W_embed, W_lin_s[0], wrs[0], wrg[0])
    rvcur = jnp.zeros((NP, 3 * M1), f32)

    w0 = _w_stage(evt_p, Wm1[0].T, Wm2[0].T, Wm3[0].T)
    wlist = [w0, None]

    for l in range(L):
        g = _gather(t, send_p).reshape(EP // 4, 128)
        q = _edge_stage(evt_p, g, wlist[l])
        if l == 0:
            wlist[1] = _w_stage(evt_p, Wm1[1].T, Wm2[1].T, Wm3[1].T)
        agg = _scatter(q.reshape(NC, NG, R, PW), recv_p, zeros_tab)
        if l < L - 1:
            t, rcur, rvcur = _node_update_mid(
                agg, rcur, rvcur, spec_p, W_out_s[l], W_out_v[l],
                W_lin_s[l + 1], W_lin_v[l + 1],
                wrs[l + 1], wrg[l + 1], wrv[l + 1])
        else:
            e = _node_update_last(agg, rcur, rvcur, W_out_s[l], W_out_v[l],
                                  W_f1, W_f2)
    return e[:N, 0]
